# Initial kernel scaffold; baseline (speedup 1.0000x reference)
import jax
import jax.numpy as jnp
from jax.experimental import pallas as pl


def kernel(indices, W):
    raise NotImplementedError("write your pallas kernel here")


# speedup vs baseline: 4.9456x; 4.9456x over previous
import functools

import jax
import jax.numpy as jnp
from jax import lax
from jax.experimental import pallas as pl
from jax.experimental.pallas import tpu as pltpu
from jax.experimental.pallas import tpu_sc as plsc

BATCH = 16384
HIST = 200
EMBED = 32
TOTAL = BATCH * HIST
NUM_CORES = 2
NUM_SUBCORES = 16
NW = NUM_CORES * NUM_SUBCORES
PER_W = TOTAL // NW
CHUNK = 2048
NCHUNK = PER_W // CHUNK


def _embedding_body(table_hbm, idx_hbm, out_hbm, idx_v, rows_v, sem):
    wid = lax.axis_index("s") * NUM_CORES + lax.axis_index("c")
    base = wid * PER_W

    def step(c, carry):
        off = base + c * CHUNK
        pltpu.sync_copy(idx_hbm.at[pl.ds(off, CHUNK)], idx_v)
        pltpu.async_copy(table_hbm.at[idx_v], rows_v, sem).wait()
        pltpu.sync_copy(rows_v, out_hbm.at[pl.ds(off, CHUNK)])
        return carry

    lax.fori_loop(0, NCHUNK, step, 0)


def kernel(indices, W):
    flat = indices.reshape(TOTAL).astype(jnp.int32)
    mesh = plsc.VectorSubcoreMesh(core_axis_name="c", subcore_axis_name="s")
    run = functools.partial(
        pl.kernel,
        mesh=mesh,
        out_type=jax.ShapeDtypeStruct((TOTAL, EMBED), jnp.float32),
        scratch_types=[
            pltpu.VMEM((CHUNK,), jnp.int32),
            pltpu.VMEM((CHUNK, EMBED), jnp.float32),
            pltpu.SemaphoreType.DMA,
        ],
        compiler_params=pltpu.CompilerParams(use_tc_tiling_on_sc=False),
    )(_embedding_body)
    out = run(W, flat)
    return out.reshape(BATCH, HIST, EMBED)


# speedup vs baseline: 5.0561x; 1.0223x over previous
import functools

import jax
import jax.numpy as jnp
from jax import lax
from jax.experimental import pallas as pl
from jax.experimental.pallas import tpu as pltpu
from jax.experimental.pallas import tpu_sc as plsc

BATCH = 16384
HIST = 200
EMBED = 32
TOTAL = BATCH * HIST
NUM_CORES = 2
NUM_SUBCORES = 16
NW = NUM_CORES * NUM_SUBCORES
PER_W = TOTAL // NW
CHUNK = 800
NCHUNK = PER_W // CHUNK
NBUF = 4
LOOK = 2
NGROUP = NCHUNK // NBUF


def _embedding_body(table_hbm, idx_hbm, out_hbm, idx_v, rows_v, gsem, ssem):
    wid = lax.axis_index("s") * NUM_CORES + lax.axis_index("c")
    base = wid * PER_W

    def issue_gather(n, b):
        off = base + n * CHUNK
        pltpu.sync_copy(idx_hbm.at[pl.ds(off, CHUNK)], idx_v.at[b])
        pltpu.async_copy(table_hbm.at[idx_v.at[b]], rows_v.at[b], gsem.at[b])

    def wait_gather(b):
        pltpu.make_async_copy(
            table_hbm.at[idx_v.at[b]], rows_v.at[b], gsem.at[b]).wait()

    def issue_store(c, b):
        off = base + c * CHUNK
        pltpu.async_copy(rows_v.at[b], out_hbm.at[pl.ds(off, CHUNK)],
                         ssem.at[b])

    def wait_store(b):
        pltpu.make_async_copy(
            rows_v.at[b], out_hbm.at[pl.ds(base, CHUNK)], ssem.at[b]).wait()

    def chunk_step(c, b, first_lap):
        n = c + LOOK
        bn = (b + LOOK) % NBUF
        if not (first_lap and b < NBUF - LOOK):
            wait_store(bn)
        issue_gather(n, bn)
        wait_gather(b)
        issue_store(c, b)

    for n in range(LOOK):
        issue_gather(n, n)

    for b in range(NBUF):
        chunk_step(b, b, first_lap=True)

    def group(g, carry):
        for b in range(NBUF):
            chunk_step(g * NBUF + b, b, first_lap=False)
        return carry

    lax.fori_loop(1, NGROUP - 1, group, 0)

    gl = NGROUP - 1
    for b in range(NBUF):
        c = gl * NBUF + b
        if b < NBUF - LOOK:
            bn = (b + LOOK) % NBUF
            wait_store(bn)
            issue_gather(c + LOOK, bn)
        wait_gather(b)
        issue_store(c, b)

    for b in range(NBUF):
        wait_store(b)


def kernel(indices, W):
    flat = indices.reshape(TOTAL).astype(jnp.int32)
    mesh = plsc.VectorSubcoreMesh(core_axis_name="c", subcore_axis_name="s")
    run = functools.partial(
        pl.kernel,
        mesh=mesh,
        out_type=jax.ShapeDtypeStruct((TOTAL, EMBED), jnp.float32),
        scratch_types=[
            pltpu.VMEM((NBUF, CHUNK), jnp.int32),
            pltpu.VMEM((NBUF, CHUNK, EMBED), jnp.float32),
            pltpu.SemaphoreType.DMA((NBUF,)),
            pltpu.SemaphoreType.DMA((NBUF,)),
        ],
        compiler_params=pltpu.CompilerParams(use_tc_tiling_on_sc=False),
    )(_embedding_body)
    out = run(W, flat)
    return out.reshape(BATCH, HIST, EMBED)


# speedup vs baseline: 5.5374x; 1.0952x over previous
import functools

import jax
import jax.numpy as jnp
from jax import lax
from jax.experimental import pallas as pl
from jax.experimental.pallas import tpu as pltpu
from jax.experimental.pallas import tpu_sc as plsc

BATCH = 16384
HIST = 200
EMBED = 32
TOTAL = BATCH * HIST
NUM_CORES = 2
NUM_SUBCORES = 16
NW = NUM_CORES * NUM_SUBCORES
PER_W = TOTAL // NW
CHUNK = 800
NCHUNK = PER_W // CHUNK
NBUF = 4
LOOK = 2
NGROUP = NCHUNK // NBUF


def _embedding_body(table_hbm, idx_hbm, out_hbm, idx_v, rows_v, gsem, ssem):
    wid = lax.axis_index("s") * NUM_CORES + lax.axis_index("c")
    base = wid * PER_W

    def issue_gather(n, b):
        off = base + n * CHUNK
        pltpu.sync_copy(idx_hbm.at[pl.ds(off, CHUNK)], idx_v.at[b])
        pltpu.async_copy(table_hbm.at[idx_v.at[b]], rows_v.at[b], gsem.at[b])

    def wait_gather(b):
        pltpu.make_async_copy(
            table_hbm.at[idx_v.at[b]], rows_v.at[b], gsem.at[b]).wait()

    def issue_store(c, b):
        off = base + c * CHUNK
        pltpu.async_copy(rows_v.at[b], out_hbm.at[pl.ds(off, CHUNK)],
                         ssem.at[b])

    def wait_store(b):
        pltpu.make_async_copy(
            rows_v.at[b], out_hbm.at[pl.ds(base, CHUNK)], ssem.at[b]).wait()

    def chunk_step(c, b, first_lap):
        n = c + LOOK
        bn = (b + LOOK) % NBUF
        if not (first_lap and b < NBUF - LOOK):
            wait_store(bn)
        issue_gather(n, bn)
        wait_gather(b)
        issue_store(c, b)

    for n in range(LOOK):
        issue_gather(n, n)

    for b in range(NBUF):
        chunk_step(b, b, first_lap=True)

    def group(g, carry):
        for b in range(NBUF):
            chunk_step(g * NBUF + b, b, first_lap=False)
        return carry

    lax.fori_loop(1, NGROUP - 1, group, 0)

    gl = NGROUP - 1
    for b in range(NBUF):
        c = gl * NBUF + b
        if b < NBUF - LOOK:
            bn = (b + LOOK) % NBUF
            wait_store(bn)
            issue_gather(c + LOOK, bn)
        wait_gather(b)
        issue_store(c, b)

    for b in range(NBUF):
        wait_store(b)


def kernel(indices, W):
    flat = indices.T.reshape(TOTAL).astype(jnp.int32)
    mesh = plsc.VectorSubcoreMesh(core_axis_name="c", subcore_axis_name="s")
    run = functools.partial(
        pl.kernel,
        mesh=mesh,
        out_type=jax.ShapeDtypeStruct((TOTAL, EMBED), jnp.float32),
        scratch_types=[
            pltpu.VMEM((NBUF, CHUNK), jnp.int32),
            pltpu.VMEM((NBUF, CHUNK, EMBED), jnp.float32),
            pltpu.SemaphoreType.DMA((NBUF,)),
            pltpu.SemaphoreType.DMA((NBUF,)),
        ],
        compiler_params=pltpu.CompilerParams(use_tc_tiling_on_sc=False),
    )(_embedding_body)
    out = run(W, flat)
    return out.reshape(HIST, BATCH, EMBED).transpose(1, 0, 2)
